# baseline (speedup 1.0000x reference)
import functools

import jax
import jax.numpy as jnp
from jax import lax
from jax.experimental import pallas as pl
from jax.experimental.pallas import tpu as pltpu
from jax.experimental.pallas import tpu_sc as plsc

XS, YS, ZS = 128, 128, 128
F4 = 16
MAX_WEIGHT = 500.0
V = XS * YS * ZS
ROWS = V // 128
N = 240 * 320

NC, NS = 2, 16
PER_TILE = N // NS
CHUNK = V // 4
NTRASH = 64
ACC_PAD = 1024
ZONE = (CHUNK + ACC_PAD) // NS
SLICE = CHUNK // NS


def _lin3(idx):
    return YS * ZS * idx[..., 0] + ZS * idx[..., 1] + idx[..., 2]


def _valid3(idx):
    return ((idx[..., 0] >= 0) & (idx[..., 0] < XS)
            & (idx[..., 1] >= 0) & (idx[..., 1] < YS)
            & (idx[..., 2] >= 0) & (idx[..., 2] < ZS))


def _split_idx(lin):
    tr = CHUNK + (jnp.arange(N, dtype=jnp.int32) & (NTRASH - 1))
    parts = []
    for k in range(4):
        base = k * CHUNK
        inr = (lin >= base) & (lin < base + CHUNK)
        parts.append(jnp.where(inr, lin - base, tr))
    return jnp.concatenate(parts).astype(jnp.int32)


def _acc_body(lin4, line4, flin4, upd, w, ones, we,
              o_upd, o_w, o_cnt, o_we, o_cnte, o_c,
              acc, zbuf, idxv0, idxv1, payv):
    core = lax.axis_index("c")
    tid = lax.axis_index("s")
    nbase = tid * PER_TILE

    def zinit(i, _):
        zbuf[pl.ds(i * 16, 16)] = jnp.zeros((16,), jnp.float32)
        return 0
    lax.fori_loop(0, ZONE // 16, zinit, 0)

    passes = ((lin4, upd, o_upd), (lin4, w, o_w), (lin4, ones, o_cnt),
              (line4, we, o_we), (line4, ones, o_cnte), (flin4, ones, o_c))
    for idx_hbm, pay_hbm, out_hbm in passes:
        pltpu.sync_copy(idx_hbm.at[pl.ds((core * 2 + 0) * N + nbase, PER_TILE)], idxv0)
        pltpu.sync_copy(idx_hbm.at[pl.ds((core * 2 + 1) * N + nbase, PER_TILE)], idxv1)
        pltpu.sync_copy(pay_hbm.at[pl.ds(nbase, PER_TILE)], payv)
        for c, idxv in ((0, idxv0), (1, idxv1)):
            pltpu.sync_copy(zbuf, acc.at[pl.ds(tid * ZONE, ZONE)])
            plsc.subcore_barrier()
            pltpu.sync_copy(payv, acc.at[idxv], add=True)
            plsc.subcore_barrier()
            pltpu.sync_copy(
                acc.at[pl.ds(tid * SLICE, SLICE)],
                out_hbm.at[pl.ds((core * 2 + c) * CHUNK + tid * SLICE, SLICE)])
            plsc.subcore_barrier()


def _sc_accumulate(lin2, line2, flin2, upd, w, ones, we):
    mesh = plsc.VectorSubcoreMesh(core_axis_name="c", subcore_axis_name="s")
    f = pl.kernel(
        _acc_body,
        out_type=[jax.ShapeDtypeStruct((V,), jnp.float32)] * 6,
        name="sc_scalar_accumulate",
        mesh=mesh,
        scratch_types=[
            pltpu.VMEM_SHARED((CHUNK + ACC_PAD,), jnp.float32),
            pltpu.VMEM((ZONE,), jnp.float32),
            pltpu.VMEM((PER_TILE,), jnp.int32),
            pltpu.VMEM((PER_TILE,), jnp.int32),
            pltpu.VMEM((PER_TILE,), jnp.float32),
        ],
    )
    return f(lin2, line2, flin2, upd, w, ones, we)


C2F = 32768
C2W = C2F * F4
NCH_F = 32
LCAP = 32
CAPB = 16 * LCAP
FPAD = 2048
FZWORDS = C2W // NS


def _feat_body(flin_hbm, feats_hbm, fd_out, facc, zbuf, flinv, posg, locidx,
               staging, stagef, widx, counts, gsem, zsem, wsem):
    core = lax.axis_index("c")
    tid = lax.axis_index("s")
    nbase = tid * PER_TILE
    lanes = lax.broadcasted_iota(jnp.int32, (16,), 0)

    def zinit(r, _):
        zbuf[pl.ds(r * 16, 16)] = jnp.zeros((16,), jnp.float32)
        return 0
    lax.fori_loop(0, FZWORDS // 16, zinit, 0)

    pltpu.sync_copy(flin_hbm.at[pl.ds(nbase, PER_TILE)], flinv)

    def binit(i, _):
        s = lanes + i * 16
        locidx[pl.ds(i * 16, 16)] = C2W + (s & 63)
        posg[pl.ds(i * 16, 16)] = (s + tid * 512) & 4095
        return 0
    lax.fori_loop(0, NCH_F * CAPB // 16, binit, 0)

    def cinit(i, _):
        counts[pl.ds(i * 16, 16)] = jnp.zeros((16,), jnp.int32)
        return 0
    lax.fori_loop(0, NCH_F, cinit, 0)

    def brec(j, _):
        gl = flinv[pl.ds(j * 16, 16)]
        gc = lax.shift_right_logical(gl, C2F.bit_length() - 1)
        b = gc & (NCH_F - 1)
        own = lax.shift_right_logical(gc, 5) == core
        cidx = b * 16 + lanes
        cnt = plsc.load_gather(counts, [cidx])
        ok = own & (cnt < LCAP)
        slot = b * CAPB + lanes * LCAP + jnp.minimum(cnt, LCAP - 1)
        loc = (lax.shift_right_logical(gl, 7) & 255) * (F4 * 128) + (gl & 127)
        plsc.store_scatter(posg, [slot], nbase + j * 16 + lanes, mask=ok)
        plsc.store_scatter(locidx, [slot], loc, mask=ok)
        plsc.store_scatter(counts, [cidx], cnt + ok.astype(jnp.int32))
        return 0
    lax.fori_loop(0, PER_TILE // 16, brec, 0)
    plsc.subcore_barrier()

    @pl.when(tid == 0)
    def _():
        pltpu.sync_copy(zbuf.at[pl.ds(0, FPAD)], facc.at[pl.ds(C2W, FPAD)])
    plsc.subcore_barrier()

    zslice = facc.at[pl.ds(tid * FZWORDS, FZWORDS)]

    def chunk(c, _):
        cb = c * CAPB
        gets = []
        for jj in range(CAPB // 16):
            pv = posg[pl.ds(cb + jj * 16, 16)]
            gets.append(pltpu.async_copy(
                feats_hbm.at[pv], staging.at[pl.ds(jj * 16, 16), :], gsem))

        @pl.when(c > 0)
        def _():
            pltpu.make_async_copy(zslice, fd_out.at[pl.ds(0, FZWORDS)],
                                  wsem).wait()
        zr = pltpu.async_copy(zbuf, zslice, zsem)
        for g in gets:
            g.wait()
        for g16 in range(CAPB // 16):
            bases = locidx[pl.ds(cb + g16 * 16, 16)]
            for k in range(16):
                rec = g16 * 16 + k
                wv = jnp.full((16,), bases[k], jnp.int32) + lanes * 128
                widx[pl.ds(rec * 16, 16)] = wv
                stagef[pl.ds(rec * 16, 16)] = staging[rec, :]
        zr.wait()
        plsc.subcore_barrier()
        pltpu.sync_copy(stagef, facc.at[widx], add=True)
        plsc.subcore_barrier()
        gbase = (core * NCH_F + c) * C2W + tid * FZWORDS
        pltpu.async_copy(zslice, fd_out.at[pl.ds(gbase, FZWORDS)], wsem)
        return 0
    lax.fori_loop(0, NCH_F, chunk, 0)
    pltpu.make_async_copy(zslice, fd_out.at[pl.ds(0, FZWORDS)], wsem).wait()
    plsc.subcore_barrier()


def _sc_feat_accumulate(flin, feats):
    mesh = plsc.VectorSubcoreMesh(core_axis_name="c", subcore_axis_name="s")
    f = pl.kernel(
        _feat_body,
        out_type=jax.ShapeDtypeStruct((V * F4,), jnp.float32),
        mesh=mesh,
        name="sc_feature_accumulate",
        compiler_params=pltpu.CompilerParams(needs_layout_passes=False,
                                             use_tc_tiling_on_sc=False),
        scratch_types=[
            pltpu.VMEM_SHARED((C2W + FPAD,), jnp.float32),
            pltpu.VMEM((FZWORDS,), jnp.float32),
            pltpu.VMEM((PER_TILE,), jnp.int32),
            pltpu.VMEM((NCH_F * CAPB,), jnp.int32),
            pltpu.VMEM((NCH_F * CAPB,), jnp.int32),
            pltpu.VMEM((CAPB, F4), jnp.float32),
            pltpu.VMEM((CAPB * F4,), jnp.float32),
            pltpu.VMEM((CAPB * F4,), jnp.int32),
            pltpu.VMEM((NCH_F * 16,), jnp.int32),
            pltpu.SemaphoreType.DMA,
            pltpu.SemaphoreType.DMA,
            pltpu.SemaphoreType.DMA,
        ],
    )
    return f(flin, feats)


def _merge_scalar_body(vold, wold, updd, wd, cntd, wed, cnted,
                       ov, ow):
    v = vold[...]
    w = wold[...]
    upd = updd[...]
    wds = wd[...]
    cnt = cntd[...]
    we = wed[...]
    cnte = cnted[...]
    touched = cnt > 0.0
    touched_e = cnte > 0.0
    num = w * v
    vu = (num + upd) / (w + wds)
    wu = jnp.clip(w + wds, 0.0, MAX_WEIGHT)
    vue = (num + we) / (w + we)
    wue = jnp.clip(w + we, 0.0, MAX_WEIGHT)
    ov[...] = jnp.where(touched_e, vue, jnp.where(touched, vu, v))
    ow[...] = jnp.where(touched_e, wue, jnp.where(touched, wu, w))


def _merge_feat_body(fwold, cd, fold, fd, ofw, of):
    fw = fwold[...]
    c = cd[...]
    ofw[...] = jnp.where(c > 0.0, jnp.clip(fw + c, 0.0, MAX_WEIGHT), fw)
    d = fw + c
    br = fw.shape[0]
    a = jnp.where(c > 0.0, fw / d, 1.0)
    b = jnp.where(c > 0.0, 1.0 / d, 0.0)
    ax = jnp.broadcast_to(a[:, None, :], (br, F4, 128)).reshape(br * F4, 128)
    bx = jnp.broadcast_to(b[:, None, :], (br, F4, 128)).reshape(br * F4, 128)
    of[...] = fold[...] * ax + fd[...] * bx


def kernel(update_values, update_features, update_indices, update_feature_indices,
           filter_indices, update_weights, update_indices_empty, update_weights_empty,
           values_volume, features_volume, weights_volume, feature_weights_volume):
    idx = update_indices.reshape(-1, 3)
    m = _valid3(idx)
    lin = jnp.where(m, _lin3(idx), V)
    w = jnp.where(m, update_weights.reshape(-1), 0.0)
    upd = w * update_values.reshape(-1)

    idxe = update_indices_empty.reshape(-1, 3)
    me = _valid3(idxe)
    line = jnp.where(me, _lin3(idxe), V)
    we = jnp.where(me, update_weights_empty.reshape(-1), 0.0)

    fidx = update_feature_indices.reshape(-1, 3)
    fm = _valid3(fidx)
    flin = jnp.where(fm, _lin3(fidx), V)

    ones = jnp.ones((N,), jnp.float32)

    upd_d, w_d, cnt_d, we_d, cnte_d, c_d = _sc_accumulate(
        _split_idx(lin), _split_idx(line), _split_idx(flin), upd, w, ones, we)

    feats = update_features.reshape(-1, F4)
    f_d = _sc_feat_accumulate(flin.astype(jnp.int32), feats)

    r2 = lambda a: a.reshape(ROWS, 128)
    f_t = features_volume.reshape(ROWS, 128, F4).transpose(0, 2, 1)
    f_t = f_t.reshape(ROWS * F4, 128)
    fd_t = f_d.reshape(ROWS * F4, 128)
    BR = 256
    grid = (ROWS // BR,)
    bs = pl.BlockSpec((BR, 128), lambda i: (i, 0))
    bf = pl.BlockSpec((BR * F4, 128), lambda i: (i, 0))
    ov, ow = pl.pallas_call(
        _merge_scalar_body,
        grid=grid,
        in_specs=[bs] * 7,
        out_specs=[bs, bs],
        out_shape=[jax.ShapeDtypeStruct((ROWS, 128), jnp.float32)] * 2,
    )(r2(values_volume), r2(weights_volume),
      r2(upd_d), r2(w_d), r2(cnt_d), r2(we_d), r2(cnte_d))
    ofw, of = pl.pallas_call(
        _merge_feat_body,
        grid=grid,
        in_specs=[bs, bs, bf, bf],
        out_specs=[bs, bf],
        out_shape=[jax.ShapeDtypeStruct((ROWS, 128), jnp.float32),
                   jax.ShapeDtypeStruct((ROWS * F4, 128), jnp.float32)],
    )(r2(feature_weights_volume), r2(c_d), f_t, fd_t)

    of4 = of.reshape(ROWS, F4, 128).transpose(0, 2, 1)
    return (ov.reshape(XS, YS, ZS), ow.reshape(XS, YS, ZS),
            of4.reshape(XS, YS, ZS, F4), ofw.reshape(XS, YS, ZS))


# speedup vs baseline: 1.0438x; 1.0438x over previous
import functools

import jax
import jax.numpy as jnp
from jax import lax
from jax.experimental import pallas as pl
from jax.experimental.pallas import tpu as pltpu
from jax.experimental.pallas import tpu_sc as plsc

XS, YS, ZS = 128, 128, 128
F4 = 16
MAX_WEIGHT = 500.0
V = XS * YS * ZS
ROWS = V // 128
N = 240 * 320

NC, NS = 2, 16
PER_TILE = N // NS
CHUNK = V // 4
NTRASH = 64
ACC_PAD = 1024
ZONE = (CHUNK + ACC_PAD) // NS
SLICE = CHUNK // NS


def _lin3(idx):
    return YS * ZS * idx[..., 0] + ZS * idx[..., 1] + idx[..., 2]


def _valid3(idx):
    return ((idx[..., 0] >= 0) & (idx[..., 0] < XS)
            & (idx[..., 1] >= 0) & (idx[..., 1] < YS)
            & (idx[..., 2] >= 0) & (idx[..., 2] < ZS))


def _split_idx(lin):
    tr = CHUNK + (jnp.arange(N, dtype=jnp.int32) & (NTRASH - 1))
    parts = []
    for k in range(4):
        base = k * CHUNK
        inr = (lin >= base) & (lin < base + CHUNK)
        parts.append(jnp.where(inr, lin - base, tr))
    return jnp.concatenate(parts).astype(jnp.int32)


def _acc_body(lin4, line4, flin4, upd, w, ones, we,
              o_upd, o_w, o_cnt, o_we, o_cnte, o_c,
              acc, zbuf, idxv0, idxv1, payv):
    core = lax.axis_index("c")
    tid = lax.axis_index("s")
    nbase = tid * PER_TILE

    def zinit(i, _):
        zbuf[pl.ds(i * 16, 16)] = jnp.zeros((16,), jnp.float32)
        return 0
    lax.fori_loop(0, ZONE // 16, zinit, 0)

    passes = ((lin4, upd, o_upd), (lin4, w, o_w), (lin4, ones, o_cnt),
              (line4, we, o_we), (line4, ones, o_cnte), (flin4, ones, o_c))
    for idx_hbm, pay_hbm, out_hbm in passes:
        pltpu.sync_copy(idx_hbm.at[pl.ds((core * 2 + 0) * N + nbase, PER_TILE)], idxv0)
        pltpu.sync_copy(idx_hbm.at[pl.ds((core * 2 + 1) * N + nbase, PER_TILE)], idxv1)
        pltpu.sync_copy(pay_hbm.at[pl.ds(nbase, PER_TILE)], payv)
        for c, idxv in ((0, idxv0), (1, idxv1)):
            pltpu.sync_copy(zbuf, acc.at[pl.ds(tid * ZONE, ZONE)])
            plsc.subcore_barrier()
            pltpu.sync_copy(payv, acc.at[idxv], add=True)
            plsc.subcore_barrier()
            pltpu.sync_copy(
                acc.at[pl.ds(tid * SLICE, SLICE)],
                out_hbm.at[pl.ds((core * 2 + c) * CHUNK + tid * SLICE, SLICE)])
            plsc.subcore_barrier()


def _sc_accumulate(lin2, line2, flin2, upd, w, ones, we):
    mesh = plsc.VectorSubcoreMesh(core_axis_name="c", subcore_axis_name="s")
    f = pl.kernel(
        _acc_body,
        out_type=[jax.ShapeDtypeStruct((V,), jnp.float32)] * 6,
        name="sc_scalar_accumulate",
        mesh=mesh,
        scratch_types=[
            pltpu.VMEM_SHARED((CHUNK + ACC_PAD,), jnp.float32),
            pltpu.VMEM((ZONE,), jnp.float32),
            pltpu.VMEM((PER_TILE,), jnp.int32),
            pltpu.VMEM((PER_TILE,), jnp.int32),
            pltpu.VMEM((PER_TILE,), jnp.float32),
        ],
    )
    return f(lin2, line2, flin2, upd, w, ones, we)


C2F = 32768
C2W = C2F * F4
NCH_F = 32
LCAP = 32
CAPB = 16 * LCAP
FPAD = 2048
FZWORDS = C2W // NS


def _feat_body(flin_hbm, feats_hbm, fd_out, facc, zbuf, flinv, posg, locidx,
               staging, stagef, widx, counts, gsem, zsem, wsem):
    core = lax.axis_index("c")
    tid = lax.axis_index("s")
    nbase = tid * PER_TILE
    lanes = lax.broadcasted_iota(jnp.int32, (16,), 0)

    def zinit(r, _):
        zbuf[pl.ds(r * 16, 16)] = jnp.zeros((16,), jnp.float32)
        return 0
    lax.fori_loop(0, FZWORDS // 16, zinit, 0)

    pltpu.sync_copy(flin_hbm.at[pl.ds(nbase, PER_TILE)], flinv)

    def binit(i, _):
        s = lanes + i * 16
        locidx[pl.ds(i * 16, 16)] = C2W + (s & 63)
        posg[pl.ds(i * 16, 16)] = (s + tid * 512) & 4095
        return 0
    lax.fori_loop(0, NCH_F * CAPB // 16, binit, 0)

    def cinit(i, _):
        counts[pl.ds(i * 16, 16)] = jnp.zeros((16,), jnp.int32)
        return 0
    lax.fori_loop(0, NCH_F, cinit, 0)

    def brec(j, _):
        gl = flinv[pl.ds(j * 16, 16)]
        gc = lax.shift_right_logical(gl, C2F.bit_length() - 1)
        b = gc & (NCH_F - 1)
        own = lax.shift_right_logical(gc, 5) == core
        cidx = b * 16 + lanes
        cnt = plsc.load_gather(counts, [cidx])
        ok = own & (cnt < LCAP)
        slot = b * CAPB + lanes * LCAP + jnp.minimum(cnt, LCAP - 1)
        loc = (lax.shift_right_logical(gl, 7) & 255) * (F4 * 128) + (gl & 127)
        plsc.store_scatter(posg, [slot], nbase + j * 16 + lanes, mask=ok)
        plsc.store_scatter(locidx, [slot], loc, mask=ok)
        plsc.store_scatter(counts, [cidx], cnt + ok.astype(jnp.int32))
        return 0
    lax.fori_loop(0, PER_TILE // 16, brec, 0)
    plsc.subcore_barrier()

    @pl.when(tid == 0)
    def _():
        pltpu.sync_copy(zbuf.at[pl.ds(0, FPAD)], facc.at[pl.ds(C2W, FPAD)])
    plsc.subcore_barrier()

    zslice = facc.at[pl.ds(tid * FZWORDS, FZWORDS)]

    def chunk(c, _):
        cb = c * CAPB
        gets = []
        for jj in range(CAPB // 16):
            pv = posg[pl.ds(cb + jj * 16, 16)]
            gets.append(pltpu.async_copy(
                feats_hbm.at[pv], staging.at[pl.ds(jj * 16, 16), :], gsem))

        @pl.when(c > 0)
        def _():
            pltpu.make_async_copy(zslice, fd_out.at[pl.ds(0, FZWORDS)],
                                  wsem).wait()
        zr = pltpu.async_copy(zbuf, zslice, zsem)
        for g in gets:
            g.wait()
        for g16 in range(CAPB // 16):
            bases = locidx[pl.ds(cb + g16 * 16, 16)]
            for k in range(16):
                rec = g16 * 16 + k
                wv = jnp.full((16,), bases[k], jnp.int32) + lanes * 128
                widx[pl.ds(rec * 16, 16)] = wv
                stagef[pl.ds(rec * 16, 16)] = staging[rec, :]
        zr.wait()
        plsc.subcore_barrier()
        pltpu.sync_copy(stagef, facc.at[widx], add=True)
        plsc.subcore_barrier()
        gbase = (core * NCH_F + c) * C2W + tid * FZWORDS
        pltpu.async_copy(zslice, fd_out.at[pl.ds(gbase, FZWORDS)], wsem)
        return 0
    lax.fori_loop(0, NCH_F, chunk, 0)
    pltpu.make_async_copy(zslice, fd_out.at[pl.ds(0, FZWORDS)], wsem).wait()
    plsc.subcore_barrier()


def _sc_feat_accumulate(flin, feats):
    mesh = plsc.VectorSubcoreMesh(core_axis_name="c", subcore_axis_name="s")
    f = pl.kernel(
        _feat_body,
        out_type=jax.ShapeDtypeStruct((V * F4,), jnp.float32),
        mesh=mesh,
        name="sc_feature_accumulate",
        compiler_params=pltpu.CompilerParams(needs_layout_passes=False,
                                             use_tc_tiling_on_sc=False),
        scratch_types=[
            pltpu.VMEM_SHARED((C2W + FPAD,), jnp.float32),
            pltpu.VMEM((FZWORDS,), jnp.float32),
            pltpu.VMEM((PER_TILE,), jnp.int32),
            pltpu.VMEM((NCH_F * CAPB,), jnp.int32),
            pltpu.VMEM((NCH_F * CAPB,), jnp.int32),
            pltpu.VMEM((CAPB, F4), jnp.float32),
            pltpu.VMEM((CAPB * F4,), jnp.float32),
            pltpu.VMEM((CAPB * F4,), jnp.int32),
            pltpu.VMEM((NCH_F * 16,), jnp.int32),
            pltpu.SemaphoreType.DMA,
            pltpu.SemaphoreType.DMA,
            pltpu.SemaphoreType.DMA,
        ],
    )
    return f(flin, feats)


def _merge_body(vold, wold, fwold, updd, wd, cntd, wed, cnted, cd,
                fold, fd, ov, ow, ofw, of):
    v = vold[...]
    w = wold[...]
    fw = fwold[...]
    c = cd[...]
    upd = updd[...]
    wds = wd[...]
    cnt = cntd[...]
    we = wed[...]
    cnte = cnted[...]
    touched = cnt > 0.0
    touched_e = cnte > 0.0
    num = w * v
    vu = (num + upd) / (w + wds)
    wu = jnp.clip(w + wds, 0.0, MAX_WEIGHT)
    vue = (num + we) / (w + we)
    wue = jnp.clip(w + we, 0.0, MAX_WEIGHT)
    ov[...] = jnp.where(touched_e, vue, jnp.where(touched, vu, v))
    ow[...] = jnp.where(touched_e, wue, jnp.where(touched, wu, w))
    ofw[...] = jnp.where(c > 0.0, jnp.clip(fw + c, 0.0, MAX_WEIGHT), fw)
    d = fw + c
    br = fw.shape[0]
    a = jnp.where(c > 0.0, fw / d, 1.0)
    b = jnp.where(c > 0.0, 1.0 / d, 0.0)
    ax = jnp.broadcast_to(a[:, None, :], (br, F4, 128)).reshape(br * F4, 128)
    bx = jnp.broadcast_to(b[:, None, :], (br, F4, 128)).reshape(br * F4, 128)
    of[...] = fold[...] * ax + fd[...] * bx


def kernel(update_values, update_features, update_indices, update_feature_indices,
           filter_indices, update_weights, update_indices_empty, update_weights_empty,
           values_volume, features_volume, weights_volume, feature_weights_volume):
    idx = update_indices.reshape(-1, 3)
    m = _valid3(idx)
    lin = jnp.where(m, _lin3(idx), V)
    w = jnp.where(m, update_weights.reshape(-1), 0.0)
    upd = w * update_values.reshape(-1)

    idxe = update_indices_empty.reshape(-1, 3)
    me = _valid3(idxe)
    line = jnp.where(me, _lin3(idxe), V)
    we = jnp.where(me, update_weights_empty.reshape(-1), 0.0)

    fidx = update_feature_indices.reshape(-1, 3)
    fm = _valid3(fidx)
    flin = jnp.where(fm, _lin3(fidx), V)

    ones = jnp.ones((N,), jnp.float32)

    upd_d, w_d, cnt_d, we_d, cnte_d, c_d = _sc_accumulate(
        _split_idx(lin), _split_idx(line), _split_idx(flin), upd, w, ones, we)

    feats = update_features.reshape(-1, F4)
    f_d = _sc_feat_accumulate(flin.astype(jnp.int32), feats)

    r2 = lambda a: a.reshape(ROWS, 128)
    f_t = features_volume.reshape(ROWS, 128, F4).transpose(0, 2, 1)
    f_t = f_t.reshape(ROWS * F4, 128)
    fd_t = f_d.reshape(ROWS * F4, 128)
    BR = 256
    grid = (ROWS // BR,)
    bs = pl.BlockSpec((BR, 128), lambda i: (i, 0))
    bf = pl.BlockSpec((BR * F4, 128), lambda i: (i, 0))
    ov, ow, ofw, of = pl.pallas_call(
        _merge_body,
        grid=grid,
        in_specs=[bs] * 9 + [bf, bf],
        out_specs=[bs, bs, bs, bf],
        out_shape=[jax.ShapeDtypeStruct((ROWS, 128), jnp.float32)] * 3
        + [jax.ShapeDtypeStruct((ROWS * F4, 128), jnp.float32)],
    )(r2(values_volume), r2(weights_volume), r2(feature_weights_volume),
      r2(upd_d), r2(w_d), r2(cnt_d), r2(we_d), r2(cnte_d), r2(c_d),
      f_t, fd_t)

    of4 = of.reshape(ROWS, F4, 128).transpose(0, 2, 1)
    return (ov.reshape(XS, YS, ZS), ow.reshape(XS, YS, ZS),
            of4.reshape(XS, YS, ZS, F4), ofw.reshape(XS, YS, ZS))


# speedup vs baseline: 1.0556x; 1.0114x over previous
import functools

import jax
import jax.numpy as jnp
from jax import lax
from jax.experimental import pallas as pl
from jax.experimental.pallas import tpu as pltpu
from jax.experimental.pallas import tpu_sc as plsc

XS, YS, ZS = 128, 128, 128
F4 = 16
MAX_WEIGHT = 500.0
V = XS * YS * ZS
ROWS = V // 128
N = 240 * 320

NC, NS = 2, 16
PER_TILE = N // NS
CHUNK = V // 4
NTRASH = 64
ACC_PAD = 1024
ZONE = (CHUNK + ACC_PAD) // NS
SLICE = CHUNK // NS


def _lin3(idx):
    return YS * ZS * idx[..., 0] + ZS * idx[..., 1] + idx[..., 2]


def _valid3(idx):
    return ((idx[..., 0] >= 0) & (idx[..., 0] < XS)
            & (idx[..., 1] >= 0) & (idx[..., 1] < YS)
            & (idx[..., 2] >= 0) & (idx[..., 2] < ZS))


def _split_idx(lin):
    tr = CHUNK + (jnp.arange(N, dtype=jnp.int32) & (NTRASH - 1))
    parts = []
    for k in range(4):
        base = k * CHUNK
        inr = (lin >= base) & (lin < base + CHUNK)
        parts.append(jnp.where(inr, lin - base, tr))
    return jnp.concatenate(parts).astype(jnp.int32)


def _acc_body(lin4, line4, flin4, upd, w, ones, we,
              o_upd, o_w, o_cnt, o_we, o_cnte, o_c,
              acc, zbuf, idxvs, payvs, zsem, wsem):
    core = lax.axis_index("c")
    tid = lax.axis_index("s")
    nbase = tid * PER_TILE

    def zinit(i, _):
        zbuf[pl.ds(i * 16, 16)] = jnp.zeros((16,), jnp.float32)
        return 0
    lax.fori_loop(0, SLICE // 16, zinit, 0)

    for s, idx_hbm in enumerate((lin4, line4, flin4)):
        for c in (0, 1):
            pltpu.sync_copy(
                idx_hbm.at[pl.ds((core * 2 + c) * N + nbase, PER_TILE)],
                idxvs[s * 2 + c])
    for p, pay_hbm in enumerate((upd, w, ones, we)):
        pltpu.sync_copy(pay_hbm.at[pl.ds(nbase, PER_TILE)], payvs[p])

    @pl.when(tid == 0)
    def _():
        pltpu.sync_copy(zbuf.at[pl.ds(0, ACC_PAD)], acc.at[pl.ds(CHUNK, ACC_PAD)])

    zslice = acc.at[pl.ds(tid * SLICE, SLICE)]
    plan = ((0, payvs[0], o_upd), (0, payvs[1], o_w), (0, payvs[2], o_cnt),
            (1, payvs[3], o_we), (1, payvs[2], o_cnte), (2, payvs[2], o_c))
    first = True
    for s, payv, out_hbm in plan:
        for c in (0, 1):
            obase = (core * 2 + c) * CHUNK + tid * SLICE
            if not first:
                pltpu.make_async_copy(
                    zslice, out_hbm.at[pl.ds(obase, SLICE)], wsem).wait()
            first = False
            pltpu.async_copy(zbuf, zslice, zsem).wait()
            plsc.subcore_barrier()
            pltpu.sync_copy(payv, acc.at[idxvs[s * 2 + c]], add=True)
            plsc.subcore_barrier()
            pltpu.async_copy(zslice, out_hbm.at[pl.ds(obase, SLICE)], wsem)
    pltpu.make_async_copy(zslice, o_c.at[pl.ds(0, SLICE)], wsem).wait()
    plsc.subcore_barrier()


def _sc_accumulate(lin2, line2, flin2, upd, w, ones, we):
    mesh = plsc.VectorSubcoreMesh(core_axis_name="c", subcore_axis_name="s")
    f = pl.kernel(
        _acc_body,
        out_type=[jax.ShapeDtypeStruct((V,), jnp.float32)] * 6,
        name="sc_scalar_accumulate",
        mesh=mesh,
        scratch_types=[
            pltpu.VMEM_SHARED((CHUNK + ACC_PAD,), jnp.float32),
            pltpu.VMEM((SLICE,), jnp.float32),
            [pltpu.VMEM((PER_TILE,), jnp.int32)] * 6,
            [pltpu.VMEM((PER_TILE,), jnp.float32)] * 4,
            pltpu.SemaphoreType.DMA,
            pltpu.SemaphoreType.DMA,
        ],
    )
    return f(lin2, line2, flin2, upd, w, ones, we)


C2F = 32768
C2W = C2F * F4
NCH_F = 32
LCAP = 32
CAPB = 16 * LCAP
FPAD = 2048
FZWORDS = C2W // NS


def _feat_body(flin_hbm, feats_hbm, fd_out, facc, zbuf, flinv, posg, locidx,
               staging, stagef, widx, counts, gsem, zsem, wsem):
    core = lax.axis_index("c")
    tid = lax.axis_index("s")
    nbase = tid * PER_TILE
    lanes = lax.broadcasted_iota(jnp.int32, (16,), 0)

    def zinit(r, _):
        zbuf[pl.ds(r * 16, 16)] = jnp.zeros((16,), jnp.float32)
        return 0
    lax.fori_loop(0, FZWORDS // 16, zinit, 0)

    pltpu.sync_copy(flin_hbm.at[pl.ds(nbase, PER_TILE)], flinv)

    def binit(i, _):
        s = lanes + i * 16
        locidx[pl.ds(i * 16, 16)] = C2W + (s & 63)
        posg[pl.ds(i * 16, 16)] = (s + tid * 512) & 4095
        return 0
    lax.fori_loop(0, NCH_F * CAPB // 16, binit, 0)

    def cinit(i, _):
        counts[pl.ds(i * 16, 16)] = jnp.zeros((16,), jnp.int32)
        return 0
    lax.fori_loop(0, NCH_F, cinit, 0)

    def brec(j, _):
        gl = flinv[pl.ds(j * 16, 16)]
        gc = lax.shift_right_logical(gl, C2F.bit_length() - 1)
        b = gc & (NCH_F - 1)
        own = lax.shift_right_logical(gc, 5) == core
        cidx = b * 16 + lanes
        cnt = plsc.load_gather(counts, [cidx])
        ok = own & (cnt < LCAP)
        slot = b * CAPB + lanes * LCAP + jnp.minimum(cnt, LCAP - 1)
        loc = (lax.shift_right_logical(gl, 7) & 255) * (F4 * 128) + (gl & 127)
        plsc.store_scatter(posg, [slot], nbase + j * 16 + lanes, mask=ok)
        plsc.store_scatter(locidx, [slot], loc, mask=ok)
        plsc.store_scatter(counts, [cidx], cnt + ok.astype(jnp.int32))
        return 0
    lax.fori_loop(0, PER_TILE // 16, brec, 0)
    plsc.subcore_barrier()

    @pl.when(tid == 0)
    def _():
        pltpu.sync_copy(zbuf.at[pl.ds(0, FPAD)], facc.at[pl.ds(C2W, FPAD)])
    plsc.subcore_barrier()

    zslice = facc.at[pl.ds(tid * FZWORDS, FZWORDS)]

    def chunk(c, _):
        cb = c * CAPB
        gets = []
        for jj in range(CAPB // 16):
            pv = posg[pl.ds(cb + jj * 16, 16)]
            gets.append(pltpu.async_copy(
                feats_hbm.at[pv], staging.at[pl.ds(jj * 16, 16), :], gsem))

        @pl.when(c > 0)
        def _():
            pltpu.make_async_copy(zslice, fd_out.at[pl.ds(0, FZWORDS)],
                                  wsem).wait()
        zr = pltpu.async_copy(zbuf, zslice, zsem)
        for g in gets:
            g.wait()
        for g16 in range(CAPB // 16):
            bases = locidx[pl.ds(cb + g16 * 16, 16)]
            for k in range(16):
                rec = g16 * 16 + k
                wv = jnp.full((16,), bases[k], jnp.int32) + lanes * 128
                widx[pl.ds(rec * 16, 16)] = wv
                stagef[pl.ds(rec * 16, 16)] = staging[rec, :]
        zr.wait()
        plsc.subcore_barrier()
        pltpu.sync_copy(stagef, facc.at[widx], add=True)
        plsc.subcore_barrier()
        gbase = (core * NCH_F + c) * C2W + tid * FZWORDS
        pltpu.async_copy(zslice, fd_out.at[pl.ds(gbase, FZWORDS)], wsem)
        return 0
    lax.fori_loop(0, NCH_F, chunk, 0)
    pltpu.make_async_copy(zslice, fd_out.at[pl.ds(0, FZWORDS)], wsem).wait()
    plsc.subcore_barrier()


def _sc_feat_accumulate(flin, feats):
    mesh = plsc.VectorSubcoreMesh(core_axis_name="c", subcore_axis_name="s")
    f = pl.kernel(
        _feat_body,
        out_type=jax.ShapeDtypeStruct((V * F4,), jnp.float32),
        mesh=mesh,
        name="sc_feature_accumulate",
        compiler_params=pltpu.CompilerParams(needs_layout_passes=False,
                                             use_tc_tiling_on_sc=False),
        scratch_types=[
            pltpu.VMEM_SHARED((C2W + FPAD,), jnp.float32),
            pltpu.VMEM((FZWORDS,), jnp.float32),
            pltpu.VMEM((PER_TILE,), jnp.int32),
            pltpu.VMEM((NCH_F * CAPB,), jnp.int32),
            pltpu.VMEM((NCH_F * CAPB,), jnp.int32),
            pltpu.VMEM((CAPB, F4), jnp.float32),
            pltpu.VMEM((CAPB * F4,), jnp.float32),
            pltpu.VMEM((CAPB * F4,), jnp.int32),
            pltpu.VMEM((NCH_F * 16,), jnp.int32),
            pltpu.SemaphoreType.DMA,
            pltpu.SemaphoreType.DMA,
            pltpu.SemaphoreType.DMA,
        ],
    )
    return f(flin, feats)


def _merge_body(vold, wold, fwold, updd, wd, cntd, wed, cnted, cd,
                fold, fd, ov, ow, ofw, of):
    v = vold[...]
    w = wold[...]
    fw = fwold[...]
    c = cd[...]
    upd = updd[...]
    wds = wd[...]
    cnt = cntd[...]
    we = wed[...]
    cnte = cnted[...]
    touched = cnt > 0.0
    touched_e = cnte > 0.0
    num = w * v
    vu = (num + upd) / (w + wds)
    wu = jnp.clip(w + wds, 0.0, MAX_WEIGHT)
    vue = (num + we) / (w + we)
    wue = jnp.clip(w + we, 0.0, MAX_WEIGHT)
    ov[...] = jnp.where(touched_e, vue, jnp.where(touched, vu, v))
    ow[...] = jnp.where(touched_e, wue, jnp.where(touched, wu, w))
    ofw[...] = jnp.where(c > 0.0, jnp.clip(fw + c, 0.0, MAX_WEIGHT), fw)
    d = fw + c
    br = fw.shape[0]
    a = jnp.where(c > 0.0, fw / d, 1.0)
    b = jnp.where(c > 0.0, 1.0 / d, 0.0)
    ax = jnp.broadcast_to(a[:, None, :], (br, F4, 128)).reshape(br * F4, 128)
    bx = jnp.broadcast_to(b[:, None, :], (br, F4, 128)).reshape(br * F4, 128)
    of[...] = fold[...] * ax + fd[...] * bx


def kernel(update_values, update_features, update_indices, update_feature_indices,
           filter_indices, update_weights, update_indices_empty, update_weights_empty,
           values_volume, features_volume, weights_volume, feature_weights_volume):
    idx = update_indices.reshape(-1, 3)
    m = _valid3(idx)
    lin = jnp.where(m, _lin3(idx), V)
    w = jnp.where(m, update_weights.reshape(-1), 0.0)
    upd = w * update_values.reshape(-1)

    idxe = update_indices_empty.reshape(-1, 3)
    me = _valid3(idxe)
    line = jnp.where(me, _lin3(idxe), V)
    we = jnp.where(me, update_weights_empty.reshape(-1), 0.0)

    fidx = update_feature_indices.reshape(-1, 3)
    fm = _valid3(fidx)
    flin = jnp.where(fm, _lin3(fidx), V)

    ones = jnp.ones((N,), jnp.float32)

    upd_d, w_d, cnt_d, we_d, cnte_d, c_d = _sc_accumulate(
        _split_idx(lin), _split_idx(line), _split_idx(flin), upd, w, ones, we)

    feats = update_features.reshape(-1, F4)
    f_d = _sc_feat_accumulate(flin.astype(jnp.int32), feats)

    r2 = lambda a: a.reshape(ROWS, 128)
    f_t = features_volume.reshape(ROWS, 128, F4).transpose(0, 2, 1)
    f_t = f_t.reshape(ROWS * F4, 128)
    fd_t = f_d.reshape(ROWS * F4, 128)
    BR = 256
    grid = (ROWS // BR,)
    bs = pl.BlockSpec((BR, 128), lambda i: (i, 0))
    bf = pl.BlockSpec((BR * F4, 128), lambda i: (i, 0))
    ov, ow, ofw, of = pl.pallas_call(
        _merge_body,
        grid=grid,
        in_specs=[bs] * 9 + [bf, bf],
        out_specs=[bs, bs, bs, bf],
        out_shape=[jax.ShapeDtypeStruct((ROWS, 128), jnp.float32)] * 3
        + [jax.ShapeDtypeStruct((ROWS * F4, 128), jnp.float32)],
    )(r2(values_volume), r2(weights_volume), r2(feature_weights_volume),
      r2(upd_d), r2(w_d), r2(cnt_d), r2(we_d), r2(cnte_d), r2(c_d),
      f_t, fd_t)

    of4 = of.reshape(ROWS, F4, 128).transpose(0, 2, 1)
    return (ov.reshape(XS, YS, ZS), ow.reshape(XS, YS, ZS),
            of4.reshape(XS, YS, ZS, F4), ofw.reshape(XS, YS, ZS))
